# baseline (speedup 1.0000x reference)
import functools

import jax
import jax.numpy as jnp
from jax.experimental import pallas as pl
from jax.experimental.pallas import tpu as pltpu

B, N, H, F, FE = 8, 256, 128, 128, 16
NB_STEPS = 2
TI = 32
NI = N // TI
JP = 8
NJH = N // JP


def _mpnn_body(x_ref, e_ref, adj_ref, h0_ref, W1_ref, W2_ref, W3_ref,
               Wt_ref, W4_ref, out_ref, h_sc, srcp_sc, dst_sc, m3_sc):
    s = pl.program_id(0)
    b = pl.program_id(1)
    i = pl.program_id(2)

    @pl.when((s == 0) & (i == 0))
    def _init():
        h_sc[b] = h0_ref[0]

    @pl.when(i == 0)
    def _per_batch():
        x = x_ref[0]
        h = h_sc[b]
        msrc = (jnp.dot(x, W1_ref[:F], preferred_element_type=jnp.float32)
                + jnp.dot(h, W1_ref[F:], preferred_element_type=jnp.float32))
        srcp_sc[...] = msrc.reshape(NJH, JP * H)
        dst_sc[...] = (jnp.dot(x, W2_ref[:F], preferred_element_type=jnp.float32)
                       + jnp.dot(h, W2_ref[F:], preferred_element_type=jnp.float32))
        m3_sc[...] = (jnp.dot(x, W3_ref[:F], preferred_element_type=jnp.float32)
                      + jnp.dot(h, W3_ref[F:], preferred_element_type=jnp.float32))

    ep = e_ref[0].reshape(TI * NJH, JP * FE)
    me = jnp.dot(ep, Wt_ref[...], preferred_element_type=jnp.float32)
    me = me.reshape(TI, NJH, JP * H)

    src = srcp_sc[...]
    dstrow = dst_sc[pl.ds(i * TI, TI), :]
    dstt = jnp.concatenate([dstrow] * JP, axis=1)
    msg = jnp.maximum(me + src[None, :, :] + dstt[:, None, :], 0.0)

    a = adj_ref[0]
    ar = a.reshape(TI, NJH, JP)
    ssum = msg[:, :, 0:H] * ar[:, :, 0:1]
    for k in range(1, JP):
        ssum = ssum + msg[:, :, k * H:(k + 1) * H] * ar[:, :, k:k + 1]
    agg = jnp.sum(ssum, axis=1)

    m3 = m3_sc[pl.ds(i * TI, TI), :]
    hn = jnp.maximum(
        m3 + jnp.dot(agg, W4_ref[...], preferred_element_type=jnp.float32), 0.0)
    out_ref[0] = hn
    h_sc[b, pl.ds(i * TI, TI), :] = hn


@jax.jit
def kernel(node_fts, edge_fts, adj, hidden, W1, W2, We, W3, W4):
    e_packed = edge_fts.reshape(B, N, NJH, JP * FE)
    Wt = jnp.zeros((JP * FE, JP * H), dtype=jnp.float32)
    for k in range(JP):
        Wt = Wt.at[k * FE:(k + 1) * FE, k * H:(k + 1) * H].set(We)

    grid = (NB_STEPS, B, NI)
    out = pl.pallas_call(
        _mpnn_body,
        grid=grid,
        in_specs=[
            pl.BlockSpec((1, N, F), lambda s, b, i: (b, 0, 0)),
            pl.BlockSpec((1, TI, NJH, JP * FE), lambda s, b, i: (b, i, 0, 0)),
            pl.BlockSpec((1, TI, N), lambda s, b, i: (b, i, 0)),
            pl.BlockSpec((1, N, H), lambda s, b, i: (b, 0, 0)),
            pl.BlockSpec((F + H, H), lambda s, b, i: (0, 0)),
            pl.BlockSpec((F + H, H), lambda s, b, i: (0, 0)),
            pl.BlockSpec((F + H, H), lambda s, b, i: (0, 0)),
            pl.BlockSpec((JP * FE, JP * H), lambda s, b, i: (0, 0)),
            pl.BlockSpec((H, H), lambda s, b, i: (0, 0)),
        ],
        out_specs=pl.BlockSpec((1, TI, H), lambda s, b, i: (b, i, 0)),
        out_shape=jax.ShapeDtypeStruct((B, N, H), jnp.float32),
        scratch_shapes=[
            pltpu.VMEM((B, N, H), jnp.float32),
            pltpu.VMEM((NJH, JP * H), jnp.float32),
            pltpu.VMEM((N, H), jnp.float32),
            pltpu.VMEM((N, H), jnp.float32),
        ],
        compiler_params=pltpu.CompilerParams(
            dimension_semantics=("arbitrary", "arbitrary", "arbitrary"),
        ),
    )(node_fts, e_packed, adj, hidden, W1, W2, W3, Wt, W4)
    return out


# speedup vs baseline: 1.0737x; 1.0737x over previous
import functools

import jax
import jax.numpy as jnp
from jax.experimental import pallas as pl
from jax.experimental.pallas import tpu as pltpu

B, N, H, F, FE = 8, 256, 128, 128, 16
NB_STEPS = 2
TI = 32
NI = N // TI
JP = 8
NJH = N // JP


def _mpnn_body(x_ref, e_ref, adj_ref, h0_ref, W1_ref, W2_ref, W3_ref,
               Wt_ref, W4_ref, out_ref, h_sc, srcp_sc, dst_sc, m3_sc):
    s = pl.program_id(0)
    b = pl.program_id(1)
    i = pl.program_id(2)

    @pl.when((s == 0) & (i == 0))
    def _init():
        h_sc[b] = h0_ref[0]

    @pl.when(i == 0)
    def _per_batch():
        x = x_ref[0]
        h = h_sc[b]
        msrc = (jnp.dot(x, W1_ref[:F], preferred_element_type=jnp.float32)
                + jnp.dot(h, W1_ref[F:], preferred_element_type=jnp.float32))
        srcp_sc[...] = msrc.reshape(NJH, JP * H)
        dst_sc[...] = (jnp.dot(x, W2_ref[:F], preferred_element_type=jnp.float32)
                       + jnp.dot(h, W2_ref[F:], preferred_element_type=jnp.float32))
        m3_sc[...] = (jnp.dot(x, W3_ref[:F], preferred_element_type=jnp.float32)
                      + jnp.dot(h, W3_ref[F:], preferred_element_type=jnp.float32))

    ep = e_ref[0].reshape(TI * NJH, JP * FE)
    me = jnp.dot(ep, Wt_ref[...], preferred_element_type=jnp.float32)
    me = me.reshape(TI, NJH, JP * H)

    src = srcp_sc[...]
    dstrow = dst_sc[pl.ds(i * TI, TI), :]
    dstt = jnp.concatenate([dstrow] * JP, axis=1)
    msg = jnp.maximum(me + src[None, :, :] + dstt[:, None, :], 0.0)

    a = adj_ref[0]
    ar = a.reshape(TI, NJH, JP)
    ssum = msg[:, :, 0:H] * ar[:, :, 0:1]
    for k in range(1, JP):
        ssum = ssum + msg[:, :, k * H:(k + 1) * H] * ar[:, :, k:k + 1]
    agg = jnp.sum(ssum, axis=1)

    m3 = m3_sc[pl.ds(i * TI, TI), :]
    hn = jnp.maximum(
        m3 + jnp.dot(agg, W4_ref[...], preferred_element_type=jnp.float32), 0.0)
    out_ref[0] = hn
    h_sc[b, pl.ds(i * TI, TI), :] = hn


@jax.jit
def kernel(node_fts, edge_fts, adj, hidden, W1, W2, We, W3, W4):
    e_packed = edge_fts.reshape(B, N, NJH, JP * FE).astype(jnp.bfloat16)
    Wt = jnp.zeros((JP * FE, JP * H), dtype=jnp.float32)
    for k in range(JP):
        Wt = Wt.at[k * FE:(k + 1) * FE, k * H:(k + 1) * H].set(We)
    Wt = Wt.astype(jnp.bfloat16)

    grid = (NB_STEPS, B, NI)
    out = pl.pallas_call(
        _mpnn_body,
        grid=grid,
        in_specs=[
            pl.BlockSpec((1, N, F), lambda s, b, i: (b, 0, 0)),
            pl.BlockSpec((1, TI, NJH, JP * FE), lambda s, b, i: (b, i, 0, 0)),
            pl.BlockSpec((1, TI, N), lambda s, b, i: (b, i, 0)),
            pl.BlockSpec((1, N, H), lambda s, b, i: (b, 0, 0)),
            pl.BlockSpec((F + H, H), lambda s, b, i: (0, 0)),
            pl.BlockSpec((F + H, H), lambda s, b, i: (0, 0)),
            pl.BlockSpec((F + H, H), lambda s, b, i: (0, 0)),
            pl.BlockSpec((JP * FE, JP * H), lambda s, b, i: (0, 0)),
            pl.BlockSpec((H, H), lambda s, b, i: (0, 0)),
        ],
        out_specs=pl.BlockSpec((1, TI, H), lambda s, b, i: (b, i, 0)),
        out_shape=jax.ShapeDtypeStruct((B, N, H), jnp.float32),
        scratch_shapes=[
            pltpu.VMEM((B, N, H), jnp.float32),
            pltpu.VMEM((NJH, JP * H), jnp.float32),
            pltpu.VMEM((N, H), jnp.float32),
            pltpu.VMEM((N, H), jnp.float32),
        ],
        compiler_params=pltpu.CompilerParams(
            dimension_semantics=("arbitrary", "arbitrary", "arbitrary"),
        ),
    )(node_fts, e_packed, adj, hidden, W1, W2, W3, Wt, W4)
    return out
